# baseline (speedup 1.0000x reference)
import functools

import jax
import jax.numpy as jnp
from jax.experimental import pallas as pl
from jax.experimental.pallas import tpu as pltpu

SOLVER_ITERS = 100
KNN = 32

HIGH = jax.lax.Precision.HIGHEST


def _build_kernel(f1_ref, f2_ref, x1_ref, x2t_ref, K_ref, Kb_ref, att_ref, asum_ref):
    t = pl.program_id(1)
    f1 = f1_ref[0]
    f2 = f2_ref[0]
    sn = f1 / jnp.maximum(jnp.sqrt(jnp.sum(f1 * f1, axis=0, keepdims=True)), 1e-12)
    tn = f2 / jnp.maximum(jnp.sqrt(jnp.sum(f2 * f2, axis=0, keepdims=True)), 1e-12)
    f_sim = jax.lax.dot_general(sn, tn, (((0,), (0,)), ((), ())),
                                preferred_element_type=jnp.float32)
    x1 = x1_ref[0]
    x2t = x2t_ref[0]
    n1sq = jnp.sum(x1 * x1, axis=1, keepdims=True)
    n2sq = jnp.sum(x2t * x2t, axis=0, keepdims=True)
    e = jax.lax.dot_general(x1, x2t, (((1,), (0,)), ((), ())),
                            preferred_element_type=jnp.float32)
    d2 = (n1sq + n2sq) - 2.0 * e
    g_sim = jnp.sqrt(jnp.maximum(d2, 1e-12))
    cost = jnp.clip(1.0 - f_sim + 0.1 * g_sim, 0.0, 1.0)
    K = jnp.exp(-cost / 0.1)
    K_ref[0] = K
    Kb_ref[0] = K.astype(jnp.bfloat16)
    t_avg = jnp.mean(f2, axis=1, keepdims=True)
    att = jax.lax.dot_general(t_avg, f1, (((0,), (0,)), ((), ())),
                              preferred_element_type=jnp.float32)
    att = jnp.maximum(att, 0.0)
    att_ref[0] = att.reshape(att.shape[1], 1)
    @pl.when(t == 0)
    def _():
        asum_ref[...] = jnp.zeros_like(asum_ref)
    asum_ref[...] += jnp.sum(att, axis=1, keepdims=True).reshape(1, 1, 1)


def _proj_kernel(w1a_ref, w1b_ref, w1f_ref, x2t_ref, bc2t_ref, f2_ref,
                 phi_ref, plo_ref):
    p = jax.lax.dot_general(w1a_ref[...], x2t_ref[0], (((1,), (0,)), ((), ())),
                            preferred_element_type=jnp.float32)
    p += jax.lax.dot_general(w1b_ref[...], bc2t_ref[0], (((1,), (0,)), ((), ())),
                             preferred_element_type=jnp.float32)
    p += jax.lax.dot_general(w1f_ref[...], f2_ref[0], (((1,), (0,)), ((), ())),
                             preferred_element_type=jnp.float32)
    hi = p.astype(jnp.bfloat16)
    phi_ref[0] = hi
    plo_ref[0] = (p - hi.astype(jnp.float32)).astype(jnp.bfloat16)


def _sinkhorn_kernel(kb_hbm, att_ref, asum_ref, r_ref, c_ref,
                     kb_vmem, sem, c_s, z_s, r_s, u_s, *, n_iter, tr):
    b = pl.program_id(0)
    copy = pltpu.make_async_copy(kb_hbm.at[b], kb_vmem, sem)
    copy.start()
    u_s[...] = att_ref[0] / (asum_ref[0] + 1e-6)
    c_s[...] = jnp.ones_like(c_s)
    copy.wait()
    n1, n2 = kb_vmem.shape
    nt = n1 // tr
    vv = 1.0 / n2

    def iter_body(carry):
        i, _ = carry
        c_old = c_s[...]
        cb = c_old.astype(jnp.bfloat16).astype(jnp.float32)
        z_s[...] = jnp.zeros_like(z_s)

        def tile_body(t, acc):
            Kt = kb_vmem[pl.ds(t * tr, tr), :].astype(jnp.float32)
            y = jnp.sum(Kt * cb, axis=1, keepdims=True)
            r_t = u_s[pl.ds(t * tr, tr), :] / y
            r_s[pl.ds(t * tr, tr), :] = r_t
            rb = r_t.astype(jnp.bfloat16).astype(jnp.float32)
            z_s[...] += jnp.sum(Kt * rb, axis=0, keepdims=True)
            return acc

        jax.lax.fori_loop(0, nt, tile_body, 0)
        c_new = vv / z_s[...]
        c_s[...] = c_new
        done = jnp.all(c_new == c_old)
        return i + 1, done

    def iter_cond(carry):
        i, done = carry
        return jnp.logical_and(i < n_iter, jnp.logical_not(done))

    jax.lax.while_loop(iter_cond, iter_body, (0, False))
    r_ref[0] = r_s[...]
    c_ref[0] = c_s[...]


def _select_kernel(K_ref, r_ref, c_ref, phi_ref, plo_ref,
                   w1c_ref, g1_ref, b1_ref, w2_ref, g2_ref, b2_ref,
                   wo_ref, bo_ref, out_ref, tw_s, hmax_s, *, n2, knn):
    rows = tw_s.shape[0]
    T = jnp.clip(r_ref[0] * c_ref[0] * K_ref[0], 1e-7, 1.0)
    tw_s[...] = T
    hmax_s[...] = jnp.zeros_like(hmax_s)
    iota = jax.lax.broadcasted_iota(jnp.int32, (rows, n2), 1)
    phi = phi_ref[0]
    plo = plo_ref[0]
    w1c = w1c_ref[...].astype(jnp.bfloat16).astype(jnp.float32)
    g1 = g1_ref[...]
    b1 = b1_ref[...]
    w2 = w2_ref[...]
    g2 = g2_ref[...]
    b2 = b2_ref[...]

    def body(_, carry):
        cur = tw_s[...]
        m = jnp.max(cur, axis=1, keepdims=True)
        sel = jnp.where(cur == m, iota, n2)
        am = jnp.min(sel, axis=1, keepdims=True)
        tw_s[...] = jnp.where(iota == am, 0.0, cur)
        oh = (iota == am).astype(jnp.bfloat16)
        feat = jax.lax.dot_general(oh, phi, (((1,), (1,)), ((), ())),
                                   preferred_element_type=jnp.float32)
        feat += jax.lax.dot_general(oh, plo, (((1,), (1,)), ((), ())),
                                    preferred_element_type=jnp.float32)
        mb = m.astype(jnp.bfloat16).astype(jnp.float32)
        pre1 = feat + mb * w1c
        h1 = jnp.maximum(g1 * pre1 + b1, 0.0)
        h2 = jax.lax.dot_general(h1, w2, (((1,), (1,)), ((), ())),
                                 preferred_element_type=jnp.float32)
        h2 = jnp.maximum(g2 * h2 + b2, 0.0)
        hmax_s[...] = jnp.maximum(hmax_s[...], h2)
        return carry

    jax.lax.fori_loop(0, knn, body, 0)
    out = jax.lax.dot_general(wo_ref[...], hmax_s[...], (((1,), (1,)), ((), ())),
                              preferred_element_type=jnp.float32)
    out_ref[0] = out + bo_ref[...]


def kernel(fmap1, fmap2, xyz1, xyz2, bc1, bc2, W1, g1, b1, W2, g2, b2, W_out, b_out):
    B, C, n1 = fmap1.shape
    n2 = fmap2.shape[2]
    f32 = jnp.float32

    xyz2t = jnp.transpose(xyz2, (0, 2, 1))
    bc2t = jnp.transpose(bc2, (0, 2, 1))

    RA = 512 if n1 % 512 == 0 else n1
    nta = n1 // RA
    K, Kb, att3, asum = pl.pallas_call(
        _build_kernel,
        grid=(B, nta),
        in_specs=[
            pl.BlockSpec((1, C, RA), lambda b, t: (b, 0, t)),
            pl.BlockSpec((1, C, n2), lambda b, t: (b, 0, 0)),
            pl.BlockSpec((1, RA, 3), lambda b, t: (b, t, 0)),
            pl.BlockSpec((1, 3, n2), lambda b, t: (b, 0, 0)),
        ],
        out_specs=[
            pl.BlockSpec((1, RA, n2), lambda b, t: (b, t, 0)),
            pl.BlockSpec((1, RA, n2), lambda b, t: (b, t, 0)),
            pl.BlockSpec((1, RA, 1), lambda b, t: (b, t, 0)),
            pl.BlockSpec((1, 1, 1), lambda b, t: (b, 0, 0)),
        ],
        out_shape=[
            jax.ShapeDtypeStruct((B, n1, n2), f32),
            jax.ShapeDtypeStruct((B, n1, n2), jnp.bfloat16),
            jax.ShapeDtypeStruct((B, n1, 1), f32),
            jax.ShapeDtypeStruct((B, 1, 1), f32),
        ],
        compiler_params=pltpu.CompilerParams(
            dimension_semantics=("parallel", "arbitrary")),
    )(fmap1, fmap2, xyz1, xyz2t)
    phi, plo = pl.pallas_call(
        _proj_kernel,
        grid=(B,),
        in_specs=[
            pl.BlockSpec((128, 3), lambda b: (0, 0)),
            pl.BlockSpec((128, 9), lambda b: (0, 0)),
            pl.BlockSpec((128, C), lambda b: (0, 0)),
            pl.BlockSpec((1, 3, n2), lambda b: (b, 0, 0)),
            pl.BlockSpec((1, 9, n2), lambda b: (b, 0, 0)),
            pl.BlockSpec((1, C, n2), lambda b: (b, 0, 0)),
        ],
        out_specs=[
            pl.BlockSpec((1, 128, n2), lambda b: (b, 0, 0)),
            pl.BlockSpec((1, 128, n2), lambda b: (b, 0, 0)),
        ],
        out_shape=[
            jax.ShapeDtypeStruct((B, 128, n2), jnp.bfloat16),
            jax.ShapeDtypeStruct((B, 128, n2), jnp.bfloat16),
        ],
        compiler_params=pltpu.CompilerParams(
            dimension_semantics=("parallel",)),
    )(W1[:, 1:4], W1[:, 4:13], W1[:, 13:], xyz2t, bc2t, fmap2)

    RB = 512 if n1 % 512 == 0 else n1
    r3, cvec = pl.pallas_call(
        functools.partial(_sinkhorn_kernel, n_iter=SOLVER_ITERS, tr=RB),
        grid=(B,),
        in_specs=[
            pl.BlockSpec(memory_space=pl.ANY),
            pl.BlockSpec((1, n1, 1), lambda b: (b, 0, 0)),
            pl.BlockSpec((1, 1, 1), lambda b: (b, 0, 0)),
        ],
        out_specs=[
            pl.BlockSpec((1, n1, 1), lambda b: (b, 0, 0)),
            pl.BlockSpec((1, 1, n2), lambda b: (b, 0, 0)),
        ],
        out_shape=[
            jax.ShapeDtypeStruct((B, n1, 1), f32),
            jax.ShapeDtypeStruct((B, 1, n2), f32),
        ],
        scratch_shapes=[
            pltpu.VMEM((n1, n2), jnp.bfloat16),
            pltpu.SemaphoreType.DMA,
            pltpu.VMEM((1, n2), f32),
            pltpu.VMEM((1, n2), f32),
            pltpu.VMEM((n1, 1), f32),
            pltpu.VMEM((n1, 1), f32),
        ],
        compiler_params=pltpu.CompilerParams(
            dimension_semantics=("arbitrary",)),
    )(Kb, att3, asum)

    RC = 256 if n1 % 256 == 0 else n1
    ntc = n1 // RC
    out = pl.pallas_call(
        functools.partial(_select_kernel, n2=n2, knn=KNN),
        grid=(B, ntc),
        in_specs=[
            pl.BlockSpec((1, RC, n2), lambda b, t: (b, t, 0)),
            pl.BlockSpec((1, RC, 1), lambda b, t: (b, t, 0)),
            pl.BlockSpec((1, 1, n2), lambda b, t: (b, 0, 0)),
            pl.BlockSpec((1, 128, n2), lambda b, t: (b, 0, 0)),
            pl.BlockSpec((1, 128, n2), lambda b, t: (b, 0, 0)),
            pl.BlockSpec((1, 128), lambda b, t: (0, 0)),
            pl.BlockSpec((1, 128), lambda b, t: (0, 0)),
            pl.BlockSpec((1, 128), lambda b, t: (0, 0)),
            pl.BlockSpec((256, 128), lambda b, t: (0, 0)),
            pl.BlockSpec((1, 256), lambda b, t: (0, 0)),
            pl.BlockSpec((1, 256), lambda b, t: (0, 0)),
            pl.BlockSpec((32, 256), lambda b, t: (0, 0)),
            pl.BlockSpec((32, 1), lambda b, t: (0, 0)),
        ],
        out_specs=pl.BlockSpec((1, 32, RC), lambda b, t: (b, 0, t)),
        out_shape=jax.ShapeDtypeStruct((B, 32, n1), f32),
        scratch_shapes=[
            pltpu.VMEM((RC, n2), f32),
            pltpu.VMEM((RC, 256), f32),
        ],
        compiler_params=pltpu.CompilerParams(
            dimension_semantics=("parallel", "arbitrary")),
    )(K, r3, cvec, phi, plo,
      W1[:, 0].reshape(1, 128), g1.reshape(1, 128), b1.reshape(1, 128),
      W2, g2.reshape(1, 256), b2.reshape(1, 256),
      W_out, b_out.reshape(32, 1))

    return out


# speedup vs baseline: 1.5025x; 1.5025x over previous
import functools

import jax
import jax.numpy as jnp
import numpy as np
from jax.experimental import pallas as pl
from jax.experimental.pallas import tpu as pltpu

SOLVER_ITERS = 100
KNN = 32

HIGH = jax.lax.Precision.HIGHEST


def _build_kernel(f1_ref, f2_ref, x1_ref, x2t_ref, K_ref, Kb_ref, att_ref, asum_ref):
    t = pl.program_id(1)
    f1 = f1_ref[0]
    f2 = f2_ref[0]
    sn = f1 / jnp.maximum(jnp.sqrt(jnp.sum(f1 * f1, axis=0, keepdims=True)), 1e-12)
    tn = f2 / jnp.maximum(jnp.sqrt(jnp.sum(f2 * f2, axis=0, keepdims=True)), 1e-12)
    f_sim = jax.lax.dot_general(sn, tn, (((0,), (0,)), ((), ())),
                                preferred_element_type=jnp.float32)
    x1 = x1_ref[0]
    x2t = x2t_ref[0]
    n1sq = jnp.sum(x1 * x1, axis=1, keepdims=True)
    n2sq = jnp.sum(x2t * x2t, axis=0, keepdims=True)
    e = jax.lax.dot_general(x1, x2t, (((1,), (0,)), ((), ())),
                            preferred_element_type=jnp.float32)
    d2 = (n1sq + n2sq) - 2.0 * e
    g_sim = jnp.sqrt(jnp.maximum(d2, 1e-12))
    cost = jnp.clip(1.0 - f_sim + 0.1 * g_sim, 0.0, 1.0)
    K = jnp.exp(-cost / 0.1)
    K_ref[0] = K
    Kb_ref[0] = K.astype(jnp.bfloat16)
    t_avg = jnp.mean(f2, axis=1, keepdims=True)
    att = jax.lax.dot_general(t_avg, f1, (((0,), (0,)), ((), ())),
                              preferred_element_type=jnp.float32)
    att = jnp.maximum(att, 0.0)
    att_ref[0] = att.reshape(att.shape[1], 1)
    @pl.when(t == 0)
    def _():
        asum_ref[...] = jnp.zeros_like(asum_ref)
    asum_ref[...] += jnp.sum(att, axis=1, keepdims=True).reshape(1, 1, 1)


def _proj_kernel(w1a_ref, w1b_ref, w1f_ref, x2t_ref, bc2t_ref, f2_ref,
                 phi_ref, plo_ref):
    p = jax.lax.dot_general(w1a_ref[...], x2t_ref[0], (((1,), (0,)), ((), ())),
                            preferred_element_type=jnp.float32)
    p += jax.lax.dot_general(w1b_ref[...], bc2t_ref[0], (((1,), (0,)), ((), ())),
                             preferred_element_type=jnp.float32)
    p += jax.lax.dot_general(w1f_ref[...], f2_ref[0], (((1,), (0,)), ((), ())),
                             preferred_element_type=jnp.float32)
    hi = p.astype(jnp.bfloat16)
    phi_ref[0] = hi
    plo_ref[0] = (p - hi.astype(jnp.float32)).astype(jnp.bfloat16)


def _sinkhorn_kernel(kb_hbm, att_ref, asum_ref, r_ref, c_ref,
                     kb_vmem, sem, c_s, z_s, r_s, u_s, *, n_iter, tr):
    b = pl.program_id(0)
    copy = pltpu.make_async_copy(kb_hbm.at[b], kb_vmem, sem)
    copy.start()
    u_s[...] = att_ref[0] / (asum_ref[0] + 1e-6)
    c_s[...] = jnp.ones_like(c_s)
    copy.wait()
    n1, n2 = kb_vmem.shape
    nt = n1 // tr
    vv = 1.0 / n2

    def iter_body(carry):
        i, _ = carry
        c_old = c_s[...]
        cb = c_old.astype(jnp.bfloat16).astype(jnp.float32)
        z_s[...] = jnp.zeros_like(z_s)

        def tile_body(t, acc):
            Kt = kb_vmem[pl.ds(t * tr, tr), :].astype(jnp.float32)
            y = jnp.sum(Kt * cb, axis=1, keepdims=True)
            r_t = u_s[pl.ds(t * tr, tr), :] / y
            r_s[pl.ds(t * tr, tr), :] = r_t
            rb = r_t.astype(jnp.bfloat16).astype(jnp.float32)
            z_s[...] += jnp.sum(Kt * rb, axis=0, keepdims=True)
            return acc

        jax.lax.fori_loop(0, nt, tile_body, 0)
        c_new = vv / z_s[...]
        c_s[...] = c_new
        done = jnp.all(c_new == c_old)
        return i + 1, done

    def iter_cond(carry):
        i, done = carry
        return jnp.logical_and(i < n_iter, jnp.logical_not(done))

    jax.lax.while_loop(iter_cond, iter_body, (0, False))
    r_ref[0] = r_s[...]
    c_ref[0] = c_s[...]


def _select_kernel(K_ref, r_ref, c_ref, phi_ref, plo_ref,
                   w1c_ref, g1_ref, b1_ref, w2_ref, g2_ref, b2_ref,
                   wo_ref, bo_ref, out_ref, tw_s, hmax_s, *, n2, knn):
    rows = tw_s.shape[0]
    T = jnp.clip(r_ref[0] * c_ref[0] * K_ref[0], 1e-7, 1.0)
    tw_s[...] = T
    hmax_s[...] = jnp.zeros_like(hmax_s)
    iota = jax.lax.broadcasted_iota(jnp.int32, (rows, n2), 1)
    phi = phi_ref[0]
    plo = plo_ref[0]
    w1c = w1c_ref[...].astype(jnp.bfloat16).astype(jnp.float32)
    g1 = g1_ref[...]
    b1 = b1_ref[...]
    w2 = w2_ref[...]
    g2 = g2_ref[...]
    b2 = b2_ref[...]

    def body(_, carry):
        cur = tw_s[...]
        m = jnp.max(cur, axis=1, keepdims=True)
        sel = jnp.where(cur == m, iota, n2)
        am = jnp.min(sel, axis=1, keepdims=True)
        tw_s[...] = jnp.where(iota == am, 0.0, cur)
        oh = (iota == am).astype(jnp.bfloat16)
        feat = jax.lax.dot_general(oh, phi, (((1,), (1,)), ((), ())),
                                   preferred_element_type=jnp.float32)
        feat += jax.lax.dot_general(oh, plo, (((1,), (1,)), ((), ())),
                                    preferred_element_type=jnp.float32)
        mb = m.astype(jnp.bfloat16).astype(jnp.float32)
        pre1 = feat + mb * w1c
        h1 = jnp.maximum(g1 * pre1 + b1, 0.0)
        h2 = jax.lax.dot_general(h1, w2, (((1,), (1,)), ((), ())),
                                 preferred_element_type=jnp.float32)
        h2 = jnp.maximum(g2 * h2 + b2, 0.0)
        hmax_s[...] = jnp.maximum(hmax_s[...], h2)
        return carry

    jax.lax.fori_loop(0, knn, body, 0)
    out = jax.lax.dot_general(wo_ref[...], hmax_s[...], (((1,), (1,)), ((), ())),
                              preferred_element_type=jnp.float32)
    out_ref[0] = out + bo_ref[...]


def _impl(fmap1, fmap2, xyz1, xyz2, bc1, bc2, W1, g1, b1, W2, g2, b2, W_out, b_out):
    B, C, n1 = fmap1.shape
    n2 = fmap2.shape[2]
    f32 = jnp.float32

    xyz2t = jnp.transpose(xyz2, (0, 2, 1))
    bc2t = jnp.transpose(bc2, (0, 2, 1))

    RA = 512 if n1 % 512 == 0 else n1
    nta = n1 // RA
    K, Kb, att3, asum = pl.pallas_call(
        _build_kernel,
        grid=(B, nta),
        in_specs=[
            pl.BlockSpec((1, C, RA), lambda b, t: (b, 0, t)),
            pl.BlockSpec((1, C, n2), lambda b, t: (b, 0, 0)),
            pl.BlockSpec((1, RA, 3), lambda b, t: (b, t, 0)),
            pl.BlockSpec((1, 3, n2), lambda b, t: (b, 0, 0)),
        ],
        out_specs=[
            pl.BlockSpec((1, RA, n2), lambda b, t: (b, t, 0)),
            pl.BlockSpec((1, RA, n2), lambda b, t: (b, t, 0)),
            pl.BlockSpec((1, RA, 1), lambda b, t: (b, t, 0)),
            pl.BlockSpec((1, 1, 1), lambda b, t: (b, 0, 0)),
        ],
        out_shape=[
            jax.ShapeDtypeStruct((B, n1, n2), f32),
            jax.ShapeDtypeStruct((B, n1, n2), jnp.bfloat16),
            jax.ShapeDtypeStruct((B, n1, 1), f32),
            jax.ShapeDtypeStruct((B, 1, 1), f32),
        ],
        compiler_params=pltpu.CompilerParams(
            dimension_semantics=("parallel", "arbitrary")),
    )(fmap1, fmap2, xyz1, xyz2t)
    phi, plo = pl.pallas_call(
        _proj_kernel,
        grid=(B,),
        in_specs=[
            pl.BlockSpec((128, 3), lambda b: (0, 0)),
            pl.BlockSpec((128, 9), lambda b: (0, 0)),
            pl.BlockSpec((128, C), lambda b: (0, 0)),
            pl.BlockSpec((1, 3, n2), lambda b: (b, 0, 0)),
            pl.BlockSpec((1, 9, n2), lambda b: (b, 0, 0)),
            pl.BlockSpec((1, C, n2), lambda b: (b, 0, 0)),
        ],
        out_specs=[
            pl.BlockSpec((1, 128, n2), lambda b: (b, 0, 0)),
            pl.BlockSpec((1, 128, n2), lambda b: (b, 0, 0)),
        ],
        out_shape=[
            jax.ShapeDtypeStruct((B, 128, n2), jnp.bfloat16),
            jax.ShapeDtypeStruct((B, 128, n2), jnp.bfloat16),
        ],
        compiler_params=pltpu.CompilerParams(
            dimension_semantics=("parallel",)),
    )(W1[:, 1:4], W1[:, 4:13], W1[:, 13:], xyz2t, bc2t, fmap2)

    RB = 512 if n1 % 512 == 0 else n1
    r3, cvec = pl.pallas_call(
        functools.partial(_sinkhorn_kernel, n_iter=SOLVER_ITERS, tr=RB),
        grid=(B,),
        in_specs=[
            pl.BlockSpec(memory_space=pl.ANY),
            pl.BlockSpec((1, n1, 1), lambda b: (b, 0, 0)),
            pl.BlockSpec((1, 1, 1), lambda b: (b, 0, 0)),
        ],
        out_specs=[
            pl.BlockSpec((1, n1, 1), lambda b: (b, 0, 0)),
            pl.BlockSpec((1, 1, n2), lambda b: (b, 0, 0)),
        ],
        out_shape=[
            jax.ShapeDtypeStruct((B, n1, 1), f32),
            jax.ShapeDtypeStruct((B, 1, n2), f32),
        ],
        scratch_shapes=[
            pltpu.VMEM((n1, n2), jnp.bfloat16),
            pltpu.SemaphoreType.DMA,
            pltpu.VMEM((1, n2), f32),
            pltpu.VMEM((1, n2), f32),
            pltpu.VMEM((n1, 1), f32),
            pltpu.VMEM((n1, 1), f32),
        ],
        compiler_params=pltpu.CompilerParams(
            dimension_semantics=("arbitrary",)),
    )(Kb, att3, asum)

    RC = 256 if n1 % 256 == 0 else n1
    ntc = n1 // RC
    out = pl.pallas_call(
        functools.partial(_select_kernel, n2=n2, knn=KNN),
        grid=(B, ntc),
        in_specs=[
            pl.BlockSpec((1, RC, n2), lambda b, t: (b, t, 0)),
            pl.BlockSpec((1, RC, 1), lambda b, t: (b, t, 0)),
            pl.BlockSpec((1, 1, n2), lambda b, t: (b, 0, 0)),
            pl.BlockSpec((1, 128, n2), lambda b, t: (b, 0, 0)),
            pl.BlockSpec((1, 128, n2), lambda b, t: (b, 0, 0)),
            pl.BlockSpec((1, 128), lambda b, t: (0, 0)),
            pl.BlockSpec((1, 128), lambda b, t: (0, 0)),
            pl.BlockSpec((1, 128), lambda b, t: (0, 0)),
            pl.BlockSpec((256, 128), lambda b, t: (0, 0)),
            pl.BlockSpec((1, 256), lambda b, t: (0, 0)),
            pl.BlockSpec((1, 256), lambda b, t: (0, 0)),
            pl.BlockSpec((32, 256), lambda b, t: (0, 0)),
            pl.BlockSpec((32, 1), lambda b, t: (0, 0)),
        ],
        out_specs=pl.BlockSpec((1, 32, RC), lambda b, t: (b, 0, t)),
        out_shape=jax.ShapeDtypeStruct((B, 32, n1), f32),
        scratch_shapes=[
            pltpu.VMEM((RC, n2), f32),
            pltpu.VMEM((RC, 256), f32),
        ],
        compiler_params=pltpu.CompilerParams(
            dimension_semantics=("parallel", "arbitrary")),
    )(K, r3, cvec, phi, plo,
      W1[:, 0].reshape(1, 128), g1.reshape(1, 128), b1.reshape(1, 128),
      W2, g2.reshape(1, 256), b2.reshape(1, 256),
      W_out, b_out.reshape(32, 1))

    return out


def kernel(fmap1, fmap2, xyz1, xyz2, bc1, bc2, W1, g1, b1, W2, g2, b2, W_out, b_out):
    B = fmap1.shape[0]
    devs = jax.devices()
    nd = 2 if (len(devs) >= 2 and B % 2 == 0) else 1
    if nd == 1:
        return _impl(fmap1, fmap2, xyz1, xyz2, bc1, bc2,
                     W1, g1, b1, W2, g2, b2, W_out, b_out)
    mesh = jax.sharding.Mesh(np.asarray(devs[:nd]), ("d",))
    P = jax.sharding.PartitionSpec
    bat = P("d")
    rep = P()
    f = jax.shard_map(
        _impl, mesh=mesh,
        in_specs=(bat, bat, bat, bat, bat, bat,
                  rep, rep, rep, rep, rep, rep, rep, rep),
        out_specs=bat,
        check_vma=False,
    )
    return f(fmap1, fmap2, xyz1, xyz2, bc1, bc2,
             W1, g1, b1, W2, g2, b2, W_out, b_out)


# speedup vs baseline: 1.5622x; 1.0397x over previous
import functools

import jax
import jax.numpy as jnp
import numpy as np
from jax.experimental import pallas as pl
from jax.experimental.pallas import tpu as pltpu

SOLVER_ITERS = 100
KNN = 32

HIGH = jax.lax.Precision.HIGHEST


def _build_kernel(f1_ref, f2_ref, x1_ref, x2t_ref, K_ref, Kb_ref, att_ref, asum_ref):
    t = pl.program_id(1)
    f1 = f1_ref[0]
    f2 = f2_ref[0]
    sn = f1 / jnp.maximum(jnp.sqrt(jnp.sum(f1 * f1, axis=0, keepdims=True)), 1e-12)
    tn = f2 / jnp.maximum(jnp.sqrt(jnp.sum(f2 * f2, axis=0, keepdims=True)), 1e-12)
    f_sim = jax.lax.dot_general(sn, tn, (((0,), (0,)), ((), ())),
                                preferred_element_type=jnp.float32)
    x1 = x1_ref[0]
    x2t = x2t_ref[0]
    n1sq = jnp.sum(x1 * x1, axis=1, keepdims=True)
    n2sq = jnp.sum(x2t * x2t, axis=0, keepdims=True)
    e = jax.lax.dot_general(x1, x2t, (((1,), (0,)), ((), ())),
                            preferred_element_type=jnp.float32)
    d2 = (n1sq + n2sq) - 2.0 * e
    g_sim = jnp.sqrt(jnp.maximum(d2, 1e-12))
    cost = jnp.clip(1.0 - f_sim + 0.1 * g_sim, 0.0, 1.0)
    K = jnp.exp(-cost / 0.1)
    K_ref[0] = K
    Kb_ref[0] = K.astype(jnp.bfloat16)
    t_avg = jnp.mean(f2, axis=1, keepdims=True)
    att = jax.lax.dot_general(t_avg, f1, (((0,), (0,)), ((), ())),
                              preferred_element_type=jnp.float32)
    att = jnp.maximum(att, 0.0)
    att_ref[0] = att.reshape(att.shape[1], 1)
    @pl.when(t == 0)
    def _():
        asum_ref[...] = jnp.zeros_like(asum_ref)
    asum_ref[...] += jnp.sum(att, axis=1, keepdims=True).reshape(1, 1, 1)


def _proj_kernel(w1a_ref, w1b_ref, w1f_ref, x2t_ref, bc2t_ref, f2_ref,
                 phi_ref, plo_ref):
    p = jax.lax.dot_general(w1a_ref[...], x2t_ref[0], (((1,), (0,)), ((), ())),
                            preferred_element_type=jnp.float32)
    p += jax.lax.dot_general(w1b_ref[...], bc2t_ref[0], (((1,), (0,)), ((), ())),
                             preferred_element_type=jnp.float32)
    p += jax.lax.dot_general(w1f_ref[...], f2_ref[0], (((1,), (0,)), ((), ())),
                             preferred_element_type=jnp.float32)
    hi = p.astype(jnp.bfloat16)
    phi_ref[0] = hi
    plo_ref[0] = (p - hi.astype(jnp.float32)).astype(jnp.bfloat16)


def _sinkhorn_kernel(kb_hbm, att_ref, asum_ref, r_ref, c_ref,
                     kb_vmem, sem, c_s, z_s, r_s, u_s, *, n_iter, tr):
    b = pl.program_id(0)
    copy = pltpu.make_async_copy(kb_hbm.at[b], kb_vmem, sem)
    copy.start()
    u_s[...] = att_ref[0] / (asum_ref[0] + 1e-6)
    c_s[...] = jnp.ones_like(c_s)
    copy.wait()
    n1, n2 = kb_vmem.shape
    nt = n1 // tr
    vv = 1.0 / n2

    def iter_body(carry):
        i, _ = carry
        c_old = c_s[...]
        cb = c_old.astype(jnp.bfloat16).astype(jnp.float32)
        z_s[...] = jnp.zeros_like(z_s)

        def tile_body(t, acc):
            Kt = kb_vmem[pl.ds(t * tr, tr), :].astype(jnp.float32)
            y = jnp.sum(Kt * cb, axis=1, keepdims=True)
            r_t = u_s[pl.ds(t * tr, tr), :] / y
            r_s[pl.ds(t * tr, tr), :] = r_t
            rb = r_t.astype(jnp.bfloat16).astype(jnp.float32)
            z_s[...] += jnp.sum(Kt * rb, axis=0, keepdims=True)
            return acc

        jax.lax.fori_loop(0, nt, tile_body, 0)
        c_new = vv / z_s[...]
        c_s[...] = c_new
        done = jnp.all(c_new == c_old)
        return i + 1, done

    def iter_cond(carry):
        i, done = carry
        return jnp.logical_and(i < n_iter, jnp.logical_not(done))

    jax.lax.while_loop(iter_cond, iter_body, (0, False))
    r_ref[0] = r_s[...]
    c_ref[0] = c_s[...]


def _select_kernel(K_ref, r_ref, c_ref, phi_ref, plo_ref,
                   w1c_ref, g1_ref, b1_ref, w2_ref, g2_ref, b2_ref,
                   wo_ref, bo_ref, out_ref, tw_s, hmax_s, *, n2, knn):
    rows = tw_s.shape[0]
    T = jnp.clip(r_ref[0] * c_ref[0] * K_ref[0], 1e-7, 1.0)
    tw_s[...] = T
    hmax_s[...] = jnp.zeros_like(hmax_s)
    iota = jax.lax.broadcasted_iota(jnp.int32, (rows, n2), 1)
    phi = phi_ref[0]
    plo = plo_ref[0]
    w1c = w1c_ref[...].astype(jnp.bfloat16).astype(jnp.float32)
    g1 = g1_ref[...]
    b1 = b1_ref[...]
    w2 = w2_ref[...]
    g2 = g2_ref[...]
    b2 = b2_ref[...]

    def body(_, carry):
        cur = tw_s[...]
        m = jnp.max(cur, axis=1, keepdims=True)
        sel = jnp.where(cur == m, iota, n2)
        am = jnp.min(sel, axis=1, keepdims=True)
        match = sel == am
        tw_s[...] = jnp.where(match, 0.0, cur)
        oh = match.astype(jnp.bfloat16)
        feat = jax.lax.dot_general(oh, phi, (((1,), (1,)), ((), ())),
                                   preferred_element_type=jnp.float32)
        feat += jax.lax.dot_general(oh, plo, (((1,), (1,)), ((), ())),
                                    preferred_element_type=jnp.float32)
        mb = m.astype(jnp.bfloat16).astype(jnp.float32)
        pre1 = feat + mb * w1c
        h1 = jnp.maximum(g1 * pre1 + b1, 0.0)
        h2 = jax.lax.dot_general(h1, w2, (((1,), (1,)), ((), ())),
                                 preferred_element_type=jnp.float32)
        h2 = jnp.maximum(g2 * h2 + b2, 0.0)
        hmax_s[...] = jnp.maximum(hmax_s[...], h2)
        return carry

    jax.lax.fori_loop(0, knn, body, 0, unroll=2)
    out = jax.lax.dot_general(wo_ref[...], hmax_s[...], (((1,), (1,)), ((), ())),
                              preferred_element_type=jnp.float32)
    out_ref[0] = out + bo_ref[...]


def _impl(fmap1, fmap2, xyz1, xyz2, bc1, bc2, W1, g1, b1, W2, g2, b2, W_out, b_out):
    B, C, n1 = fmap1.shape
    n2 = fmap2.shape[2]
    f32 = jnp.float32

    xyz2t = jnp.transpose(xyz2, (0, 2, 1))
    bc2t = jnp.transpose(bc2, (0, 2, 1))

    RA = 512 if n1 % 512 == 0 else n1
    nta = n1 // RA
    K, Kb, att3, asum = pl.pallas_call(
        _build_kernel,
        grid=(B, nta),
        in_specs=[
            pl.BlockSpec((1, C, RA), lambda b, t: (b, 0, t)),
            pl.BlockSpec((1, C, n2), lambda b, t: (b, 0, 0)),
            pl.BlockSpec((1, RA, 3), lambda b, t: (b, t, 0)),
            pl.BlockSpec((1, 3, n2), lambda b, t: (b, 0, 0)),
        ],
        out_specs=[
            pl.BlockSpec((1, RA, n2), lambda b, t: (b, t, 0)),
            pl.BlockSpec((1, RA, n2), lambda b, t: (b, t, 0)),
            pl.BlockSpec((1, RA, 1), lambda b, t: (b, t, 0)),
            pl.BlockSpec((1, 1, 1), lambda b, t: (b, 0, 0)),
        ],
        out_shape=[
            jax.ShapeDtypeStruct((B, n1, n2), f32),
            jax.ShapeDtypeStruct((B, n1, n2), jnp.bfloat16),
            jax.ShapeDtypeStruct((B, n1, 1), f32),
            jax.ShapeDtypeStruct((B, 1, 1), f32),
        ],
        compiler_params=pltpu.CompilerParams(
            dimension_semantics=("parallel", "arbitrary")),
    )(fmap1, fmap2, xyz1, xyz2t)
    phi, plo = pl.pallas_call(
        _proj_kernel,
        grid=(B,),
        in_specs=[
            pl.BlockSpec((128, 3), lambda b: (0, 0)),
            pl.BlockSpec((128, 9), lambda b: (0, 0)),
            pl.BlockSpec((128, C), lambda b: (0, 0)),
            pl.BlockSpec((1, 3, n2), lambda b: (b, 0, 0)),
            pl.BlockSpec((1, 9, n2), lambda b: (b, 0, 0)),
            pl.BlockSpec((1, C, n2), lambda b: (b, 0, 0)),
        ],
        out_specs=[
            pl.BlockSpec((1, 128, n2), lambda b: (b, 0, 0)),
            pl.BlockSpec((1, 128, n2), lambda b: (b, 0, 0)),
        ],
        out_shape=[
            jax.ShapeDtypeStruct((B, 128, n2), jnp.bfloat16),
            jax.ShapeDtypeStruct((B, 128, n2), jnp.bfloat16),
        ],
        compiler_params=pltpu.CompilerParams(
            dimension_semantics=("parallel",)),
    )(W1[:, 1:4], W1[:, 4:13], W1[:, 13:], xyz2t, bc2t, fmap2)

    RB = 512 if n1 % 512 == 0 else n1
    r3, cvec = pl.pallas_call(
        functools.partial(_sinkhorn_kernel, n_iter=SOLVER_ITERS, tr=RB),
        grid=(B,),
        in_specs=[
            pl.BlockSpec(memory_space=pl.ANY),
            pl.BlockSpec((1, n1, 1), lambda b: (b, 0, 0)),
            pl.BlockSpec((1, 1, 1), lambda b: (b, 0, 0)),
        ],
        out_specs=[
            pl.BlockSpec((1, n1, 1), lambda b: (b, 0, 0)),
            pl.BlockSpec((1, 1, n2), lambda b: (b, 0, 0)),
        ],
        out_shape=[
            jax.ShapeDtypeStruct((B, n1, 1), f32),
            jax.ShapeDtypeStruct((B, 1, n2), f32),
        ],
        scratch_shapes=[
            pltpu.VMEM((n1, n2), jnp.bfloat16),
            pltpu.SemaphoreType.DMA,
            pltpu.VMEM((1, n2), f32),
            pltpu.VMEM((1, n2), f32),
            pltpu.VMEM((n1, 1), f32),
            pltpu.VMEM((n1, 1), f32),
        ],
        compiler_params=pltpu.CompilerParams(
            dimension_semantics=("arbitrary",)),
    )(Kb, att3, asum)

    RC = 256 if n1 % 256 == 0 else n1
    ntc = n1 // RC
    out = pl.pallas_call(
        functools.partial(_select_kernel, n2=n2, knn=KNN),
        grid=(B, ntc),
        in_specs=[
            pl.BlockSpec((1, RC, n2), lambda b, t: (b, t, 0)),
            pl.BlockSpec((1, RC, 1), lambda b, t: (b, t, 0)),
            pl.BlockSpec((1, 1, n2), lambda b, t: (b, 0, 0)),
            pl.BlockSpec((1, 128, n2), lambda b, t: (b, 0, 0)),
            pl.BlockSpec((1, 128, n2), lambda b, t: (b, 0, 0)),
            pl.BlockSpec((1, 128), lambda b, t: (0, 0)),
            pl.BlockSpec((1, 128), lambda b, t: (0, 0)),
            pl.BlockSpec((1, 128), lambda b, t: (0, 0)),
            pl.BlockSpec((256, 128), lambda b, t: (0, 0)),
            pl.BlockSpec((1, 256), lambda b, t: (0, 0)),
            pl.BlockSpec((1, 256), lambda b, t: (0, 0)),
            pl.BlockSpec((32, 256), lambda b, t: (0, 0)),
            pl.BlockSpec((32, 1), lambda b, t: (0, 0)),
        ],
        out_specs=pl.BlockSpec((1, 32, RC), lambda b, t: (b, 0, t)),
        out_shape=jax.ShapeDtypeStruct((B, 32, n1), f32),
        scratch_shapes=[
            pltpu.VMEM((RC, n2), f32),
            pltpu.VMEM((RC, 256), f32),
        ],
        compiler_params=pltpu.CompilerParams(
            dimension_semantics=("parallel", "arbitrary")),
    )(K, r3, cvec, phi, plo,
      W1[:, 0].reshape(1, 128), g1.reshape(1, 128), b1.reshape(1, 128),
      W2, g2.reshape(1, 256), b2.reshape(1, 256),
      W_out, b_out.reshape(32, 1))

    return out


def kernel(fmap1, fmap2, xyz1, xyz2, bc1, bc2, W1, g1, b1, W2, g2, b2, W_out, b_out):
    B = fmap1.shape[0]
    devs = jax.devices()
    nd = 2 if (len(devs) >= 2 and B % 2 == 0) else 1
    if nd == 1:
        return _impl(fmap1, fmap2, xyz1, xyz2, bc1, bc2,
                     W1, g1, b1, W2, g2, b2, W_out, b_out)
    mesh = jax.sharding.Mesh(np.asarray(devs[:nd]), ("d",))
    P = jax.sharding.PartitionSpec
    bat = P("d")
    rep = P()
    f = jax.shard_map(
        _impl, mesh=mesh,
        in_specs=(bat, bat, bat, bat, bat, bat,
                  rep, rep, rep, rep, rep, rep, rep, rep),
        out_specs=bat,
        check_vma=False,
    )
    return f(fmap1, fmap2, xyz1, xyz2, bc1, bc2,
             W1, g1, b1, W2, g2, b2, W_out, b_out)


# speedup vs baseline: 1.5737x; 1.0074x over previous
import functools

import jax
import jax.numpy as jnp
import numpy as np
from jax.experimental import pallas as pl
from jax.experimental.pallas import tpu as pltpu

SOLVER_ITERS = 100
KNN = 32

HIGH = jax.lax.Precision.HIGHEST


def _build_kernel(f1_ref, f2_ref, x1_ref, x2t_ref, K_ref, Kb_ref, att_ref, asum_ref):
    t = pl.program_id(1)
    f1 = f1_ref[0]
    f2 = f2_ref[0]
    sn = f1 / jnp.maximum(jnp.sqrt(jnp.sum(f1 * f1, axis=0, keepdims=True)), 1e-12)
    tn = f2 / jnp.maximum(jnp.sqrt(jnp.sum(f2 * f2, axis=0, keepdims=True)), 1e-12)
    f_sim = jax.lax.dot_general(sn, tn, (((0,), (0,)), ((), ())),
                                preferred_element_type=jnp.float32)
    x1 = x1_ref[0]
    x2t = x2t_ref[0]
    n1sq = jnp.sum(x1 * x1, axis=1, keepdims=True)
    n2sq = jnp.sum(x2t * x2t, axis=0, keepdims=True)
    e = jax.lax.dot_general(x1, x2t, (((1,), (0,)), ((), ())),
                            preferred_element_type=jnp.float32)
    d2 = (n1sq + n2sq) - 2.0 * e
    g_sim = jnp.sqrt(jnp.maximum(d2, 1e-12))
    cost = jnp.clip(1.0 - f_sim + 0.1 * g_sim, 0.0, 1.0)
    K = jnp.exp(-cost / 0.1)
    K_ref[0] = K
    Kb_ref[0] = K.astype(jnp.bfloat16)
    t_avg = jnp.mean(f2, axis=1, keepdims=True)
    att = jax.lax.dot_general(t_avg, f1, (((0,), (0,)), ((), ())),
                              preferred_element_type=jnp.float32)
    att = jnp.maximum(att, 0.0)
    att_ref[0] = att.reshape(att.shape[1], 1)
    @pl.when(t == 0)
    def _():
        asum_ref[...] = jnp.zeros_like(asum_ref)
    asum_ref[...] += jnp.sum(att, axis=1, keepdims=True).reshape(1, 1, 1)


def _proj_kernel(w1a_ref, w1b_ref, w1f_ref, x2t_ref, bc2t_ref, f2_ref,
                 phi_ref, plo_ref):
    p = jax.lax.dot_general(w1a_ref[...], x2t_ref[0], (((1,), (0,)), ((), ())),
                            preferred_element_type=jnp.float32)
    p += jax.lax.dot_general(w1b_ref[...], bc2t_ref[0], (((1,), (0,)), ((), ())),
                             preferred_element_type=jnp.float32)
    p += jax.lax.dot_general(w1f_ref[...], f2_ref[0], (((1,), (0,)), ((), ())),
                             preferred_element_type=jnp.float32)
    hi = p.astype(jnp.bfloat16)
    phi_ref[0] = hi
    plo_ref[0] = (p - hi.astype(jnp.float32)).astype(jnp.bfloat16)


def _sinkhorn_kernel(kb_hbm, att_ref, asum_ref, r_ref, c_ref,
                     kb_vmem, sem, c_s, z_s, r_s, u_s, *, n_iter, tr):
    b = pl.program_id(0)
    copy = pltpu.make_async_copy(kb_hbm.at[b], kb_vmem, sem)
    copy.start()
    u_s[...] = att_ref[0] / (asum_ref[0] + 1e-6)
    c_s[...] = jnp.ones_like(c_s)
    copy.wait()
    n1, n2 = kb_vmem.shape
    nt = n1 // tr
    vv = 1.0 / n2

    def iter_body(carry):
        i, _ = carry
        c_old = c_s[...]
        cb = c_old.astype(jnp.bfloat16).astype(jnp.float32)
        z_s[...] = jnp.zeros_like(z_s)

        def tile_body(t, acc):
            Kt = kb_vmem[pl.ds(t * tr, tr), :].astype(jnp.float32)
            y = jnp.sum(Kt * cb, axis=1, keepdims=True)
            r_t = u_s[pl.ds(t * tr, tr), :] / y
            r_s[pl.ds(t * tr, tr), :] = r_t
            rb = r_t.astype(jnp.bfloat16).astype(jnp.float32)
            z_s[...] += jnp.sum(Kt * rb, axis=0, keepdims=True)
            return acc

        jax.lax.fori_loop(0, nt, tile_body, 0)
        c_new = vv / z_s[...]
        c_s[...] = c_new
        done = jnp.all(c_new == c_old)
        return i + 1, done

    def iter_cond(carry):
        i, done = carry
        return jnp.logical_and(i < n_iter, jnp.logical_not(done))

    jax.lax.while_loop(iter_cond, iter_body, (0, False))
    r_ref[0] = r_s[...]
    c_ref[0] = c_s[...]


def _select_kernel(K_ref, r_ref, c_ref, phi_ref, plo_ref,
                   w1c_ref, g1_ref, b1_ref, w2_ref, g2_ref, b2_ref,
                   wo_ref, bo_ref, out_ref, tw_s, hmax_s, *, n2, knn):
    rows = tw_s.shape[0]
    T = jnp.clip(r_ref[0] * c_ref[0] * K_ref[0], 1e-7, 1.0)
    tw_s[...] = T
    hmax_s[...] = jnp.zeros_like(hmax_s)
    iota = jax.lax.broadcasted_iota(jnp.int32, (rows, n2), 1)
    phi = phi_ref[0]
    plo = plo_ref[0]
    w1c = w1c_ref[...].astype(jnp.bfloat16).astype(jnp.float32)
    g1 = g1_ref[...]
    b1 = b1_ref[...]
    w2 = w2_ref[...]
    g2 = g2_ref[...]
    b2 = b2_ref[...]

    def body(_, carry):
        cur = tw_s[...]
        m = jnp.max(cur, axis=1, keepdims=True)
        sel = jnp.where(cur == m, iota, n2)
        am = jnp.min(sel, axis=1, keepdims=True)
        match = sel == am
        tw_s[...] = jnp.where(match, 0.0, cur)
        oh = match.astype(jnp.bfloat16)
        feat = jax.lax.dot_general(oh, phi, (((1,), (1,)), ((), ())),
                                   preferred_element_type=jnp.float32)
        feat += jax.lax.dot_general(oh, plo, (((1,), (1,)), ((), ())),
                                    preferred_element_type=jnp.float32)
        mb = m.astype(jnp.bfloat16).astype(jnp.float32)
        pre1 = feat + mb * w1c
        h1 = jnp.maximum(g1 * pre1 + b1, 0.0)
        h2 = jax.lax.dot_general(h1, w2, (((1,), (1,)), ((), ())),
                                 preferred_element_type=jnp.float32)
        h2 = jnp.maximum(g2 * h2 + b2, 0.0)
        hmax_s[...] = jnp.maximum(hmax_s[...], h2)
        return carry

    jax.lax.fori_loop(0, knn, body, 0, unroll=4)
    out = jax.lax.dot_general(wo_ref[...], hmax_s[...], (((1,), (1,)), ((), ())),
                              preferred_element_type=jnp.float32)
    out_ref[0] = out + bo_ref[...]


def _impl(fmap1, fmap2, xyz1, xyz2, bc1, bc2, W1, g1, b1, W2, g2, b2, W_out, b_out):
    B, C, n1 = fmap1.shape
    n2 = fmap2.shape[2]
    f32 = jnp.float32

    xyz2t = jnp.transpose(xyz2, (0, 2, 1))
    bc2t = jnp.transpose(bc2, (0, 2, 1))

    RA = 512 if n1 % 512 == 0 else n1
    nta = n1 // RA
    K, Kb, att3, asum = pl.pallas_call(
        _build_kernel,
        grid=(B, nta),
        in_specs=[
            pl.BlockSpec((1, C, RA), lambda b, t: (b, 0, t)),
            pl.BlockSpec((1, C, n2), lambda b, t: (b, 0, 0)),
            pl.BlockSpec((1, RA, 3), lambda b, t: (b, t, 0)),
            pl.BlockSpec((1, 3, n2), lambda b, t: (b, 0, 0)),
        ],
        out_specs=[
            pl.BlockSpec((1, RA, n2), lambda b, t: (b, t, 0)),
            pl.BlockSpec((1, RA, n2), lambda b, t: (b, t, 0)),
            pl.BlockSpec((1, RA, 1), lambda b, t: (b, t, 0)),
            pl.BlockSpec((1, 1, 1), lambda b, t: (b, 0, 0)),
        ],
        out_shape=[
            jax.ShapeDtypeStruct((B, n1, n2), f32),
            jax.ShapeDtypeStruct((B, n1, n2), jnp.bfloat16),
            jax.ShapeDtypeStruct((B, n1, 1), f32),
            jax.ShapeDtypeStruct((B, 1, 1), f32),
        ],
        compiler_params=pltpu.CompilerParams(
            dimension_semantics=("parallel", "arbitrary")),
    )(fmap1, fmap2, xyz1, xyz2t)
    phi, plo = pl.pallas_call(
        _proj_kernel,
        grid=(B,),
        in_specs=[
            pl.BlockSpec((128, 3), lambda b: (0, 0)),
            pl.BlockSpec((128, 9), lambda b: (0, 0)),
            pl.BlockSpec((128, C), lambda b: (0, 0)),
            pl.BlockSpec((1, 3, n2), lambda b: (b, 0, 0)),
            pl.BlockSpec((1, 9, n2), lambda b: (b, 0, 0)),
            pl.BlockSpec((1, C, n2), lambda b: (b, 0, 0)),
        ],
        out_specs=[
            pl.BlockSpec((1, 128, n2), lambda b: (b, 0, 0)),
            pl.BlockSpec((1, 128, n2), lambda b: (b, 0, 0)),
        ],
        out_shape=[
            jax.ShapeDtypeStruct((B, 128, n2), jnp.bfloat16),
            jax.ShapeDtypeStruct((B, 128, n2), jnp.bfloat16),
        ],
        compiler_params=pltpu.CompilerParams(
            dimension_semantics=("parallel",)),
    )(W1[:, 1:4], W1[:, 4:13], W1[:, 13:], xyz2t, bc2t, fmap2)

    RB = 512 if n1 % 512 == 0 else n1
    r3, cvec = pl.pallas_call(
        functools.partial(_sinkhorn_kernel, n_iter=SOLVER_ITERS, tr=RB),
        grid=(B,),
        in_specs=[
            pl.BlockSpec(memory_space=pl.ANY),
            pl.BlockSpec((1, n1, 1), lambda b: (b, 0, 0)),
            pl.BlockSpec((1, 1, 1), lambda b: (b, 0, 0)),
        ],
        out_specs=[
            pl.BlockSpec((1, n1, 1), lambda b: (b, 0, 0)),
            pl.BlockSpec((1, 1, n2), lambda b: (b, 0, 0)),
        ],
        out_shape=[
            jax.ShapeDtypeStruct((B, n1, 1), f32),
            jax.ShapeDtypeStruct((B, 1, n2), f32),
        ],
        scratch_shapes=[
            pltpu.VMEM((n1, n2), jnp.bfloat16),
            pltpu.SemaphoreType.DMA,
            pltpu.VMEM((1, n2), f32),
            pltpu.VMEM((1, n2), f32),
            pltpu.VMEM((n1, 1), f32),
            pltpu.VMEM((n1, 1), f32),
        ],
        compiler_params=pltpu.CompilerParams(
            dimension_semantics=("arbitrary",)),
    )(Kb, att3, asum)

    RC = 256 if n1 % 256 == 0 else n1
    ntc = n1 // RC
    out = pl.pallas_call(
        functools.partial(_select_kernel, n2=n2, knn=KNN),
        grid=(B, ntc),
        in_specs=[
            pl.BlockSpec((1, RC, n2), lambda b, t: (b, t, 0)),
            pl.BlockSpec((1, RC, 1), lambda b, t: (b, t, 0)),
            pl.BlockSpec((1, 1, n2), lambda b, t: (b, 0, 0)),
            pl.BlockSpec((1, 128, n2), lambda b, t: (b, 0, 0)),
            pl.BlockSpec((1, 128, n2), lambda b, t: (b, 0, 0)),
            pl.BlockSpec((1, 128), lambda b, t: (0, 0)),
            pl.BlockSpec((1, 128), lambda b, t: (0, 0)),
            pl.BlockSpec((1, 128), lambda b, t: (0, 0)),
            pl.BlockSpec((256, 128), lambda b, t: (0, 0)),
            pl.BlockSpec((1, 256), lambda b, t: (0, 0)),
            pl.BlockSpec((1, 256), lambda b, t: (0, 0)),
            pl.BlockSpec((32, 256), lambda b, t: (0, 0)),
            pl.BlockSpec((32, 1), lambda b, t: (0, 0)),
        ],
        out_specs=pl.BlockSpec((1, 32, RC), lambda b, t: (b, 0, t)),
        out_shape=jax.ShapeDtypeStruct((B, 32, n1), f32),
        scratch_shapes=[
            pltpu.VMEM((RC, n2), f32),
            pltpu.VMEM((RC, 256), f32),
        ],
        compiler_params=pltpu.CompilerParams(
            dimension_semantics=("parallel", "arbitrary")),
    )(K, r3, cvec, phi, plo,
      W1[:, 0].reshape(1, 128), g1.reshape(1, 128), b1.reshape(1, 128),
      W2, g2.reshape(1, 256), b2.reshape(1, 256),
      W_out, b_out.reshape(32, 1))

    return out


def kernel(fmap1, fmap2, xyz1, xyz2, bc1, bc2, W1, g1, b1, W2, g2, b2, W_out, b_out):
    B = fmap1.shape[0]
    devs = jax.devices()
    nd = 2 if (len(devs) >= 2 and B % 2 == 0) else 1
    if nd == 1:
        return _impl(fmap1, fmap2, xyz1, xyz2, bc1, bc2,
                     W1, g1, b1, W2, g2, b2, W_out, b_out)
    mesh = jax.sharding.Mesh(np.asarray(devs[:nd]), ("d",))
    P = jax.sharding.PartitionSpec
    bat = P("d")
    rep = P()
    f = jax.shard_map(
        _impl, mesh=mesh,
        in_specs=(bat, bat, bat, bat, bat, bat,
                  rep, rep, rep, rep, rep, rep, rep, rep),
        out_specs=bat,
        check_vma=False,
    )
    return f(fmap1, fmap2, xyz1, xyz2, bc1, bc2,
             W1, g1, b1, W2, g2, b2, W_out, b_out)


# speedup vs baseline: 1.6376x; 1.0406x over previous
import functools

import jax
import jax.numpy as jnp
import numpy as np
from jax.experimental import pallas as pl
from jax.experimental.pallas import tpu as pltpu

SOLVER_ITERS = 100
KNN = 32

HIGH = jax.lax.Precision.HIGHEST


def _build_kernel(f1_ref, f2_ref, x1_ref, x2t_ref, K_ref, Kb_ref, att_ref, asum_ref):
    t = pl.program_id(1)
    f1 = f1_ref[0]
    f2 = f2_ref[0]
    sn = f1 / jnp.maximum(jnp.sqrt(jnp.sum(f1 * f1, axis=0, keepdims=True)), 1e-12)
    tn = f2 / jnp.maximum(jnp.sqrt(jnp.sum(f2 * f2, axis=0, keepdims=True)), 1e-12)
    f_sim = jax.lax.dot_general(sn, tn, (((0,), (0,)), ((), ())),
                                preferred_element_type=jnp.float32)
    x1 = x1_ref[0]
    x2t = x2t_ref[0]
    n1sq = jnp.sum(x1 * x1, axis=1, keepdims=True)
    n2sq = jnp.sum(x2t * x2t, axis=0, keepdims=True)
    e = jax.lax.dot_general(x1, x2t, (((1,), (0,)), ((), ())),
                            preferred_element_type=jnp.float32)
    d2 = (n1sq + n2sq) - 2.0 * e
    g_sim = jnp.sqrt(jnp.maximum(d2, 1e-12))
    cost = jnp.clip(1.0 - f_sim + 0.1 * g_sim, 0.0, 1.0)
    K = jnp.exp(-cost / 0.1)
    K_ref[0] = K
    Kb_ref[0] = K.astype(jnp.bfloat16)
    t_avg = jnp.mean(f2, axis=1, keepdims=True)
    att = jax.lax.dot_general(t_avg, f1, (((0,), (0,)), ((), ())),
                              preferred_element_type=jnp.float32)
    att = jnp.maximum(att, 0.0)
    att_ref[0] = att.reshape(att.shape[1], 1)
    @pl.when(t == 0)
    def _():
        asum_ref[...] = jnp.zeros_like(asum_ref)
    asum_ref[...] += jnp.sum(att, axis=1, keepdims=True).reshape(1, 1, 1)


def _proj_kernel(w1a_ref, w1b_ref, w1f_ref, x2t_ref, bc2t_ref, f2_ref,
                 pp_ref):
    p = jax.lax.dot_general(w1a_ref[...], x2t_ref[0], (((1,), (0,)), ((), ())),
                            preferred_element_type=jnp.float32)
    p += jax.lax.dot_general(w1b_ref[...], bc2t_ref[0], (((1,), (0,)), ((), ())),
                             preferred_element_type=jnp.float32)
    p += jax.lax.dot_general(w1f_ref[...], f2_ref[0], (((1,), (0,)), ((), ())),
                             preferred_element_type=jnp.float32)
    hi = p.astype(jnp.bfloat16)
    lo = (p - hi.astype(jnp.float32)).astype(jnp.bfloat16)
    pp_ref[0] = jnp.concatenate([hi, lo], axis=0)


def _sinkhorn_kernel(kb_hbm, att_ref, asum_ref, r_ref, c_ref,
                     kb_vmem, sem, c_s, z_s, r_s, u_s, *, n_iter, tr):
    b = pl.program_id(0)
    copy = pltpu.make_async_copy(kb_hbm.at[b], kb_vmem, sem)
    copy.start()
    u_s[...] = att_ref[0] / (asum_ref[0] + 1e-6)
    c_s[...] = jnp.ones_like(c_s)
    copy.wait()
    n1, n2 = kb_vmem.shape
    nt = n1 // tr
    vv = 1.0 / n2

    def iter_body(carry):
        i, _ = carry
        c_old = c_s[...]
        cb = c_old.astype(jnp.bfloat16).astype(jnp.float32)
        z_s[...] = jnp.zeros_like(z_s)

        def tile_body(t, acc):
            Kt = kb_vmem[pl.ds(t * tr, tr), :].astype(jnp.float32)
            y = jnp.sum(Kt * cb, axis=1, keepdims=True)
            r_t = u_s[pl.ds(t * tr, tr), :] / y
            r_s[pl.ds(t * tr, tr), :] = r_t
            rb = r_t.astype(jnp.bfloat16).astype(jnp.float32)
            z_s[...] += jnp.sum(Kt * rb, axis=0, keepdims=True)
            return acc

        jax.lax.fori_loop(0, nt, tile_body, 0)
        c_new = vv / z_s[...]
        c_s[...] = c_new
        done = jnp.all(c_new == c_old)
        return i + 1, done

    def iter_cond(carry):
        i, done = carry
        return jnp.logical_and(i < n_iter, jnp.logical_not(done))

    jax.lax.while_loop(iter_cond, iter_body, (0, False))
    r_ref[0] = r_s[...]
    c_ref[0] = c_s[...]


def _select_kernel(K_ref, r_ref, c_ref, pp_ref,
                   w1c_ref, g1_ref, b1_ref, w2_ref, g2_ref, b2_ref,
                   wo_ref, bo_ref, out_ref, tw_s, hmax_s, *, n2, knn):
    rows = tw_s.shape[0]
    T = jnp.clip(r_ref[0] * c_ref[0] * K_ref[0], 1e-7, 1.0)
    tw_s[...] = T
    hmax_s[...] = jnp.zeros_like(hmax_s)
    iota = jax.lax.broadcasted_iota(jnp.int32, (rows, n2), 1)
    pp = pp_ref[0]
    w1c = w1c_ref[...].astype(jnp.bfloat16).astype(jnp.float32)
    g1 = g1_ref[...]
    b1 = b1_ref[...]
    w2 = w2_ref[...]
    g2 = g2_ref[...]
    b2 = b2_ref[...]

    def body(_, carry):
        cur = tw_s[...]
        m = jnp.max(cur, axis=1, keepdims=True)
        sel = jnp.where(cur == m, iota, n2)
        am = jnp.min(sel, axis=1, keepdims=True)
        match = sel == am
        tw_s[...] = jnp.where(match, 0.0, cur)
        oh = match.astype(jnp.bfloat16)
        f2 = jax.lax.dot_general(oh, pp, (((1,), (1,)), ((), ())),
                                 preferred_element_type=jnp.float32)
        feat = f2[:, :128] + f2[:, 128:]
        mb = m.astype(jnp.bfloat16).astype(jnp.float32)
        pre1 = feat + mb * w1c
        h1 = jnp.maximum(g1 * pre1 + b1, 0.0)
        h2 = jax.lax.dot_general(h1, w2, (((1,), (1,)), ((), ())),
                                 preferred_element_type=jnp.float32)
        h2 = jnp.maximum(g2 * h2 + b2, 0.0)
        hmax_s[...] = jnp.maximum(hmax_s[...], h2)
        return carry

    jax.lax.fori_loop(0, knn, body, 0, unroll=4)
    out = jax.lax.dot_general(wo_ref[...], hmax_s[...], (((1,), (1,)), ((), ())),
                              preferred_element_type=jnp.float32)
    out_ref[0] = out + bo_ref[...]


def _impl(fmap1, fmap2, xyz1, xyz2, bc1, bc2, W1, g1, b1, W2, g2, b2, W_out, b_out):
    B, C, n1 = fmap1.shape
    n2 = fmap2.shape[2]
    f32 = jnp.float32

    xyz2t = jnp.transpose(xyz2, (0, 2, 1))
    bc2t = jnp.transpose(bc2, (0, 2, 1))

    RA = 512 if n1 % 512 == 0 else n1
    nta = n1 // RA
    K, Kb, att3, asum = pl.pallas_call(
        _build_kernel,
        grid=(B, nta),
        in_specs=[
            pl.BlockSpec((1, C, RA), lambda b, t: (b, 0, t)),
            pl.BlockSpec((1, C, n2), lambda b, t: (b, 0, 0)),
            pl.BlockSpec((1, RA, 3), lambda b, t: (b, t, 0)),
            pl.BlockSpec((1, 3, n2), lambda b, t: (b, 0, 0)),
        ],
        out_specs=[
            pl.BlockSpec((1, RA, n2), lambda b, t: (b, t, 0)),
            pl.BlockSpec((1, RA, n2), lambda b, t: (b, t, 0)),
            pl.BlockSpec((1, RA, 1), lambda b, t: (b, t, 0)),
            pl.BlockSpec((1, 1, 1), lambda b, t: (b, 0, 0)),
        ],
        out_shape=[
            jax.ShapeDtypeStruct((B, n1, n2), f32),
            jax.ShapeDtypeStruct((B, n1, n2), jnp.bfloat16),
            jax.ShapeDtypeStruct((B, n1, 1), f32),
            jax.ShapeDtypeStruct((B, 1, 1), f32),
        ],
        compiler_params=pltpu.CompilerParams(
            dimension_semantics=("parallel", "arbitrary")),
    )(fmap1, fmap2, xyz1, xyz2t)
    phiplo = pl.pallas_call(
        _proj_kernel,
        grid=(B,),
        in_specs=[
            pl.BlockSpec((128, 3), lambda b: (0, 0)),
            pl.BlockSpec((128, 9), lambda b: (0, 0)),
            pl.BlockSpec((128, C), lambda b: (0, 0)),
            pl.BlockSpec((1, 3, n2), lambda b: (b, 0, 0)),
            pl.BlockSpec((1, 9, n2), lambda b: (b, 0, 0)),
            pl.BlockSpec((1, C, n2), lambda b: (b, 0, 0)),
        ],
        out_specs=pl.BlockSpec((1, 256, n2), lambda b: (b, 0, 0)),
        out_shape=jax.ShapeDtypeStruct((B, 256, n2), jnp.bfloat16),
        compiler_params=pltpu.CompilerParams(
            dimension_semantics=("parallel",)),
    )(W1[:, 1:4], W1[:, 4:13], W1[:, 13:], xyz2t, bc2t, fmap2)

    RB = 512 if n1 % 512 == 0 else n1
    r3, cvec = pl.pallas_call(
        functools.partial(_sinkhorn_kernel, n_iter=SOLVER_ITERS, tr=RB),
        grid=(B,),
        in_specs=[
            pl.BlockSpec(memory_space=pl.ANY),
            pl.BlockSpec((1, n1, 1), lambda b: (b, 0, 0)),
            pl.BlockSpec((1, 1, 1), lambda b: (b, 0, 0)),
        ],
        out_specs=[
            pl.BlockSpec((1, n1, 1), lambda b: (b, 0, 0)),
            pl.BlockSpec((1, 1, n2), lambda b: (b, 0, 0)),
        ],
        out_shape=[
            jax.ShapeDtypeStruct((B, n1, 1), f32),
            jax.ShapeDtypeStruct((B, 1, n2), f32),
        ],
        scratch_shapes=[
            pltpu.VMEM((n1, n2), jnp.bfloat16),
            pltpu.SemaphoreType.DMA,
            pltpu.VMEM((1, n2), f32),
            pltpu.VMEM((1, n2), f32),
            pltpu.VMEM((n1, 1), f32),
            pltpu.VMEM((n1, 1), f32),
        ],
        compiler_params=pltpu.CompilerParams(
            dimension_semantics=("arbitrary",)),
    )(Kb, att3, asum)

    RC = 256 if n1 % 256 == 0 else n1
    ntc = n1 // RC
    out = pl.pallas_call(
        functools.partial(_select_kernel, n2=n2, knn=KNN),
        grid=(B, ntc),
        in_specs=[
            pl.BlockSpec((1, RC, n2), lambda b, t: (b, t, 0)),
            pl.BlockSpec((1, RC, 1), lambda b, t: (b, t, 0)),
            pl.BlockSpec((1, 1, n2), lambda b, t: (b, 0, 0)),
            pl.BlockSpec((1, 256, n2), lambda b, t: (b, 0, 0)),
            pl.BlockSpec((1, 128), lambda b, t: (0, 0)),
            pl.BlockSpec((1, 128), lambda b, t: (0, 0)),
            pl.BlockSpec((1, 128), lambda b, t: (0, 0)),
            pl.BlockSpec((256, 128), lambda b, t: (0, 0)),
            pl.BlockSpec((1, 256), lambda b, t: (0, 0)),
            pl.BlockSpec((1, 256), lambda b, t: (0, 0)),
            pl.BlockSpec((32, 256), lambda b, t: (0, 0)),
            pl.BlockSpec((32, 1), lambda b, t: (0, 0)),
        ],
        out_specs=pl.BlockSpec((1, 32, RC), lambda b, t: (b, 0, t)),
        out_shape=jax.ShapeDtypeStruct((B, 32, n1), f32),
        scratch_shapes=[
            pltpu.VMEM((RC, n2), f32),
            pltpu.VMEM((RC, 256), f32),
        ],
        compiler_params=pltpu.CompilerParams(
            dimension_semantics=("parallel", "arbitrary")),
    )(K, r3, cvec, phiplo,
      W1[:, 0].reshape(1, 128), g1.reshape(1, 128), b1.reshape(1, 128),
      W2, g2.reshape(1, 256), b2.reshape(1, 256),
      W_out, b_out.reshape(32, 1))

    return out


def kernel(fmap1, fmap2, xyz1, xyz2, bc1, bc2, W1, g1, b1, W2, g2, b2, W_out, b_out):
    B = fmap1.shape[0]
    devs = jax.devices()
    nd = 2 if (len(devs) >= 2 and B % 2 == 0) else 1
    if nd == 1:
        return _impl(fmap1, fmap2, xyz1, xyz2, bc1, bc2,
                     W1, g1, b1, W2, g2, b2, W_out, b_out)
    mesh = jax.sharding.Mesh(np.asarray(devs[:nd]), ("d",))
    P = jax.sharding.PartitionSpec
    bat = P("d")
    rep = P()
    f = jax.shard_map(
        _impl, mesh=mesh,
        in_specs=(bat, bat, bat, bat, bat, bat,
                  rep, rep, rep, rep, rep, rep, rep, rep),
        out_specs=bat,
        check_vma=False,
    )
    return f(fmap1, fmap2, xyz1, xyz2, bc1, bc2,
             W1, g1, b1, W2, g2, b2, W_out, b_out)
